# baseline (speedup 1.0000x reference)
import functools

import jax
import jax.numpy as jnp
from jax import lax
from jax.experimental import pallas as pl
from jax.experimental.pallas import tpu as pltpu
from jax.experimental.pallas import tpu_sc as plsc

_B = 512
_D = 768


def _make_gather():
    info = plsc.get_sparse_core_info()
    nc, ns = 1, info.num_subcores
    nw = nc * ns
    b_per_w = _B // nw
    assert _B % (8 * nw) == 0
    mesh = plsc.VectorSubcoreMesh(core_axis_name="c", subcore_axis_name="s",
                                  num_cores=1)

    @functools.partial(
        pl.kernel,
        mesh=mesh,
        out_type=jax.ShapeDtypeStruct((_B, _D), jnp.float32),
        scratch_types=[
            pltpu.VMEM((b_per_w,), jnp.int32),
            pltpu.VMEM((b_per_w // 4, _D), jnp.float32),
            pltpu.VMEM((b_per_w // 4, _D), jnp.float32),
            pltpu.VMEM((b_per_w // 4, _D), jnp.float32),
            pltpu.VMEM((b_per_w // 4, _D), jnp.float32),
            pltpu.SemaphoreType.DMA,
            pltpu.SemaphoreType.DMA,
            pltpu.SemaphoreType.DMA,
            pltpu.SemaphoreType.DMA,
            pltpu.SemaphoreType.DMA,
        ],
    )
    def gather_kernel(table_hbm, idx_hbm, out_hbm, idx_v,
                      rows0_v, rows1_v, rows2_v, rows3_v,
                      g0_sem, g1_sem, g2_sem, g3_sem, w_sem):
        wid = lax.axis_index("s") * nc + lax.axis_index("c")
        base = wid * b_per_w
        q = b_per_w // 4
        rows = (rows0_v, rows1_v, rows2_v, rows3_v)
        gsems = (g0_sem, g1_sem, g2_sem, g3_sem)
        pltpu.sync_copy(idx_hbm.at[pl.ds(base, b_per_w)], idx_v)
        gathers = [
            pltpu.async_copy(table_hbm.at[idx_v.at[pl.ds(i * q, q)]],
                             rows[i], gsems[i])
            for i in range(4)
        ]
        writes = []
        for i in range(4):
            gathers[i].wait()
            writes.append(pltpu.async_copy(
                rows[i], out_hbm.at[pl.ds(base + i * q, q)], w_sem))
        for w in writes:
            w.wait()

    return gather_kernel


_gather = _make_gather()


def kernel(input_ids, table, position_ids):
    idx = position_ids.reshape(_B).astype(jnp.int32)
    out = _gather(table, idx)
    return out.reshape(1, _B, _D)


# speedup vs baseline: 1.0160x; 1.0160x over previous
import functools

import jax
import jax.numpy as jnp
from jax import lax
from jax.experimental import pallas as pl
from jax.experimental.pallas import tpu as pltpu
from jax.experimental.pallas import tpu_sc as plsc

_B = 512
_D = 768


def _make_gather():
    info = plsc.get_sparse_core_info()
    nc, ns = 1, info.num_subcores
    nw = nc * ns
    b_per_w = _B // nw
    assert _B % (8 * nw) == 0
    mesh = plsc.VectorSubcoreMesh(core_axis_name="c", subcore_axis_name="s",
                                  num_cores=1)

    c0 = b_per_w // 4
    c1 = b_per_w - c0

    @functools.partial(
        pl.kernel,
        mesh=mesh,
        out_type=jax.ShapeDtypeStruct((_B, _D), jnp.float32),
        scratch_types=[
            pltpu.VMEM((b_per_w,), jnp.int32),
            pltpu.VMEM((c0, _D), jnp.float32),
            pltpu.VMEM((c1, _D), jnp.float32),
            pltpu.SemaphoreType.DMA,
            pltpu.SemaphoreType.DMA,
            pltpu.SemaphoreType.DMA,
        ],
    )
    def gather_kernel(table_hbm, idx_hbm, out_hbm, idx_v, rows0_v, rows1_v,
                      g0_sem, g1_sem, w_sem):
        wid = lax.axis_index("s") * nc + lax.axis_index("c")
        base = wid * b_per_w
        pltpu.sync_copy(idx_hbm.at[pl.ds(base, b_per_w)], idx_v)
        g0 = pltpu.async_copy(table_hbm.at[idx_v.at[pl.ds(0, c0)]],
                              rows0_v, g0_sem)
        g1 = pltpu.async_copy(table_hbm.at[idx_v.at[pl.ds(c0, c1)]],
                              rows1_v, g1_sem)
        g0.wait()
        w0 = pltpu.async_copy(rows0_v, out_hbm.at[pl.ds(base, c0)], w_sem)
        g1.wait()
        w1 = pltpu.async_copy(rows1_v, out_hbm.at[pl.ds(base + c0, c1)],
                              w_sem)
        w0.wait()
        w1.wait()

    return gather_kernel


_gather = _make_gather()


def kernel(input_ids, table, position_ids):
    idx = position_ids.reshape(_B).astype(jnp.int32)
    out = _gather(table, idx)
    return out.reshape(1, _B, _D)


# speedup vs baseline: 1.0228x; 1.0066x over previous
import functools

import jax
import jax.numpy as jnp
from jax import lax
from jax.experimental import pallas as pl
from jax.experimental.pallas import tpu as pltpu
from jax.experimental.pallas import tpu_sc as plsc

_B = 512
_D = 768


def _make_gather():
    info = plsc.get_sparse_core_info()
    nc, ns = 1, info.num_subcores
    nw = nc * ns
    b_per_w = _B // nw
    assert _B % (8 * nw) == 0
    mesh = plsc.VectorSubcoreMesh(core_axis_name="c", subcore_axis_name="s",
                                  num_cores=1)

    c0 = b_per_w // 4
    c1 = b_per_w - c0

    @functools.partial(
        pl.kernel,
        mesh=mesh,
        out_type=jax.ShapeDtypeStruct((_B, _D), jnp.float32),
        scratch_types=[
            pltpu.VMEM((b_per_w,), jnp.int32),
            pltpu.VMEM((c0, _D), jnp.float32),
            pltpu.VMEM((c1, _D), jnp.float32),
            pltpu.SemaphoreType.DMA,
            pltpu.SemaphoreType.DMA,
            pltpu.SemaphoreType.DMA,
        ],
    )
    def gather_kernel(table_hbm, idx_hbm, out_hbm, idx_v, rows0_v, rows1_v,
                      g0_sem, g1_sem, w_sem):
        wid = lax.axis_index("s") * nc + lax.axis_index("c")
        base = wid * b_per_w
        pltpu.sync_copy(idx_hbm.at[0, pl.ds(base, b_per_w)], idx_v)
        g0 = pltpu.async_copy(table_hbm.at[idx_v.at[pl.ds(0, c0)]],
                              rows0_v, g0_sem)
        g1 = pltpu.async_copy(table_hbm.at[idx_v.at[pl.ds(c0, c1)]],
                              rows1_v, g1_sem)
        g0.wait()
        w0 = pltpu.async_copy(rows0_v, out_hbm.at[pl.ds(base, c0)], w_sem)
        g1.wait()
        w1 = pltpu.async_copy(rows1_v, out_hbm.at[pl.ds(base + c0, c1)],
                              w_sem)
        w0.wait()
        w1.wait()

    return gather_kernel


_gather = _make_gather()


def kernel(input_ids, table, position_ids):
    out = _gather(table, position_ids)
    return out.reshape(1, _B, _D)
